# baseline (speedup 1.0000x reference)
import functools

import jax
import jax.numpy as jnp
from jax import lax
from jax.experimental import pallas as pl
from jax.experimental.pallas import tpu as pltpu
from jax.experimental.pallas import tpu_sc as plsc

_INPUT_DIM = 96
_K = 8192
_D = 32
_BETA = 0.3
_N_TOKENS = 16 * 576
_TILE = 256
_GRID = _N_TOKENS // _TILE


def _main_body(x_ref, w_ref, b_ref, cb_ref, csq_ref, a_ref, ones_ref,
               enc_ref, idx_ref, z_ref, cnt_ref):
    x = x_ref[...]
    w = w_ref[...]
    b = b_ref[...]
    cb = cb_ref[...]
    csq = csq_ref[...]
    a = a_ref[...]
    z = jax.lax.dot_general(x, w, (((1,), (1,)), ((), ())),
                            preferred_element_type=jnp.float32) + b
    z2 = z + z
    m2 = jax.lax.dot_general(z2, cb, (((1,), (1,)), ((), ())),
                             preferred_element_type=jnp.float32)
    d = (a + csq) - m2

    n_chunks = 4
    cw = _K // n_chunks
    col = jax.lax.broadcasted_iota(jnp.int32, d.shape, 1)
    acc_v = None
    acc_i = None
    for c in range(n_chunks):
        dc = d[:, c * cw:(c + 1) * cw]
        vc = jnp.min(dc, axis=1, keepdims=True)
        wc = jnp.min(jnp.where(dc == vc, col[:, c * cw:(c + 1) * cw], _K),
                     axis=1, keepdims=True)
        vb = vc.astype(jnp.bfloat16).astype(jnp.float32)
        if acc_v is None:
            acc_v, acc_i = vb, wc
        else:
            take = vc < acc_v
            acc_i = jnp.where(take, wc, acc_i)
            acc_v = jnp.where(take, vb, acc_v)
    idx = acc_i

    onehot = (col == idx).astype(jnp.float32)
    enc_ref[...] = onehot
    idx_ref[...] = idx
    z_ref[...] = z
    cnt_ref[...] = jax.lax.dot_general(
        ones_ref[...], onehot, (((1,), (0,)), ((), ())),
        preferred_element_type=jnp.float32)[None]


def _epilogue_body(q_ref, z_ref, cnt_ref, loss_ref, perp_ref):
    q = q_ref[...]
    z = z_ref[...]
    diff = q - z
    mse = jnp.sum(diff * diff) / jnp.float32(_N_TOKENS * _D)
    loss_ref[...] = jnp.full((1, 1), mse + jnp.float32(_BETA) * mse,
                             dtype=jnp.float32)
    cnt = jnp.sum(cnt_ref[...][:, 0, :], axis=0, keepdims=True)
    p = cnt / jnp.float32(_N_TOKENS)
    perp = jnp.exp(-jnp.sum(p * jnp.log(p + 1e-10), keepdims=True))
    perp_ref[...] = perp.reshape(1, 1)


_D_PAD = 128


def _make_gather():
    info = plsc.get_sparse_core_info()
    num_cores = info.num_cores
    _NW = num_cores * info.num_subcores
    _B_PER_W = _N_TOKENS // _NW
    mesh = plsc.VectorSubcoreMesh(core_axis_name="c", subcore_axis_name="s")

    @functools.partial(
        pl.kernel, mesh=mesh,
        out_type=jax.ShapeDtypeStruct((_N_TOKENS, _D_PAD), jnp.float32),
        scratch_types=[
            pltpu.VMEM((_B_PER_W,), jnp.int32),
            pltpu.VMEM((_B_PER_W, _D_PAD), jnp.float32),
            pltpu.SemaphoreType.DMA,
        ],
    )
    def gather_k(table_hbm, idx_hbm, out_hbm, idx_v, rows_v, sem):
        wid = lax.axis_index("s") * num_cores + lax.axis_index("c")
        base = wid * _B_PER_W
        pltpu.sync_copy(idx_hbm.at[pl.ds(base, _B_PER_W)], idx_v)
        pltpu.async_copy(table_hbm.at[idx_v], rows_v, sem).wait()
        pltpu.sync_copy(rows_v, out_hbm.at[pl.ds(base, _B_PER_W)])

    return gather_k


def kernel(x, W, b, codebook):
    B, S, _ = x.shape
    flat = x.reshape(-1, _INPUT_DIM)
    b2 = b.reshape(1, _D)

    z_x = flat @ W.T + b
    a_x = jnp.sum(z_x ** 2, axis=1, keepdims=True)
    csq_x = jnp.sum(codebook ** 2, axis=1).reshape(1, _K)

    enc, idx, z, cnt_part = pl.pallas_call(
        _main_body,
        grid=(_GRID,),
        in_specs=[
            pl.BlockSpec((_TILE, _INPUT_DIM), lambda i: (i, 0)),
            pl.BlockSpec((_D, _INPUT_DIM), lambda i: (0, 0)),
            pl.BlockSpec((1, _D), lambda i: (0, 0)),
            pl.BlockSpec((_K, _D), lambda i: (0, 0)),
            pl.BlockSpec((1, _K), lambda i: (0, 0)),
            pl.BlockSpec((_TILE, 1), lambda i: (i, 0)),
            pl.BlockSpec((1, _TILE), lambda i: (0, 0)),
        ],
        out_specs=[
            pl.BlockSpec((_TILE, _K), lambda i: (i, 0)),
            pl.BlockSpec((_TILE, 1), lambda i: (i, 0)),
            pl.BlockSpec((_TILE, _D), lambda i: (i, 0)),
            pl.BlockSpec((1, 1, _K), lambda i: (i, 0, 0)),
        ],
        out_shape=[
            jax.ShapeDtypeStruct((_N_TOKENS, _K), jnp.float32),
            jax.ShapeDtypeStruct((_N_TOKENS, 1), jnp.int32),
            jax.ShapeDtypeStruct((_N_TOKENS, _D), jnp.float32),
            jax.ShapeDtypeStruct((_GRID, 1, _K), jnp.float32),
        ],
        compiler_params=pltpu.CompilerParams(
            dimension_semantics=("parallel",),
        ),
    )(flat, W, b2, codebook, csq_x, a_x, jnp.ones((1, _TILE), jnp.float32))

    cb_pad = jnp.pad(codebook, ((0, 0), (0, _D_PAD - _D)))
    q = _make_gather()(cb_pad, idx.reshape(-1))[:, :_D]

    loss, perp = pl.pallas_call(
        _epilogue_body,
        out_shape=[
            jax.ShapeDtypeStruct((1, 1), jnp.float32),
            jax.ShapeDtypeStruct((1, 1), jnp.float32),
        ],
    )(q, z, cnt_part)

    return (loss[0, 0], q.reshape(B, S, _D), perp[0, 0], enc.reshape(B, S, _K))


# speedup vs baseline: 1.0007x; 1.0007x over previous
import functools

import jax
import jax.numpy as jnp
from jax import lax
from jax.experimental import pallas as pl
from jax.experimental.pallas import tpu as pltpu
from jax.experimental.pallas import tpu_sc as plsc

_INPUT_DIM = 96
_K = 8192
_D = 32
_BETA = 0.3
_N_TOKENS = 16 * 576
_TILE = 256
_GRID = _N_TOKENS // _TILE


def _main_body(x_ref, w_ref, b_ref, cb_ref, csq_ref, a_ref, ones_ref,
               enc_ref, idx_ref, z_ref, cnt_ref):
    x = x_ref[...]
    w = w_ref[...]
    b = b_ref[...]
    cb = cb_ref[...]
    csq = csq_ref[...]
    a = a_ref[...]
    z = jax.lax.dot_general(x, w, (((1,), (1,)), ((), ())),
                            preferred_element_type=jnp.float32) + b
    z2 = z + z
    m2 = jax.lax.dot_general(z2, cb, (((1,), (1,)), ((), ())),
                             preferred_element_type=jnp.float32)
    d = (a + csq) - m2

    n_chunks = 4
    cw = _K // n_chunks
    col = jax.lax.broadcasted_iota(jnp.int32, d.shape, 1)
    acc_v = None
    acc_i = None
    for c in range(n_chunks):
        dc = d[:, c * cw:(c + 1) * cw]
        vc = jnp.min(dc, axis=1, keepdims=True)
        wc = jnp.min(jnp.where(dc == vc, col[:, c * cw:(c + 1) * cw], _K),
                     axis=1, keepdims=True)
        vb = vc.astype(jnp.bfloat16).astype(jnp.float32)
        if acc_v is None:
            acc_v, acc_i = vb, wc
        else:
            take = vc < acc_v
            acc_i = jnp.where(take, wc, acc_i)
            acc_v = jnp.where(take, vb, acc_v)
    idx = acc_i

    onehot = (col == idx).astype(jnp.float32)
    enc_ref[...] = onehot
    idx_ref[...] = idx
    z_ref[...] = z
    cnt_ref[...] = jax.lax.dot_general(
        ones_ref[...], onehot, (((1,), (0,)), ((), ())),
        preferred_element_type=jnp.float32)[None]


def _epilogue_body(q_ref, z_ref, cnt_ref, loss_ref, perp_ref):
    q = q_ref[...]
    z = z_ref[...]
    diff = q - z
    mse = jnp.sum(diff * diff) / jnp.float32(_N_TOKENS * _D)
    loss_ref[...] = jnp.full((1, 1), mse + jnp.float32(_BETA) * mse,
                             dtype=jnp.float32)
    cnt = jnp.sum(cnt_ref[...][:, 0, :], axis=0, keepdims=True)
    p = cnt / jnp.float32(_N_TOKENS)
    perp = jnp.exp(-jnp.sum(p * jnp.log(p + 1e-10), keepdims=True))
    perp_ref[...] = perp.reshape(1, 1)


_D_PAD = 128


def _make_gather():
    info = plsc.get_sparse_core_info()
    num_cores = info.num_cores
    _NW = num_cores * info.num_subcores
    _B_PER_W = _N_TOKENS // _NW
    mesh = plsc.VectorSubcoreMesh(core_axis_name="c", subcore_axis_name="s")

    @functools.partial(
        pl.kernel, mesh=mesh,
        out_type=jax.ShapeDtypeStruct((_N_TOKENS, _D_PAD), jnp.float32),
        scratch_types=[
            pltpu.VMEM((_B_PER_W,), jnp.int32),
            pltpu.VMEM((_B_PER_W, _D_PAD), jnp.float32),
            pltpu.SemaphoreType.DMA,
        ],
    )
    def gather_k(table_hbm, idx_hbm, out_hbm, idx_v, rows_v, sem):
        wid = lax.axis_index("s") * num_cores + lax.axis_index("c")
        base = wid * _B_PER_W
        pltpu.sync_copy(idx_hbm.at[pl.ds(base, _B_PER_W)], idx_v)
        pltpu.async_copy(table_hbm.at[idx_v], rows_v, sem).wait()
        pltpu.sync_copy(rows_v, out_hbm.at[pl.ds(base, _B_PER_W)])

    return gather_k


def kernel(x, W, b, codebook):
    B, S, _ = x.shape
    flat = x.reshape(-1, _INPUT_DIM)
    b2 = b.reshape(1, _D)

    z_x = flat @ W.T + b
    a_x = jnp.sum(z_x ** 2, axis=1, keepdims=True)
    csq_x = jnp.sum(codebook ** 2, axis=1).reshape(1, _K)

    enc, idx, z, cnt_part = pl.pallas_call(
        _main_body,
        grid=(_GRID,),
        in_specs=[
            pl.BlockSpec((_TILE, _INPUT_DIM), lambda i: (i, 0)),
            pl.BlockSpec((_D, _INPUT_DIM), lambda i: (0, 0)),
            pl.BlockSpec((1, _D), lambda i: (0, 0)),
            pl.BlockSpec((_K, _D), lambda i: (0, 0)),
            pl.BlockSpec((1, _K), lambda i: (0, 0)),
            pl.BlockSpec((_TILE, 1), lambda i: (i, 0)),
            pl.BlockSpec((1, _TILE), lambda i: (0, 0)),
        ],
        out_specs=[
            pl.BlockSpec((_TILE, _K), lambda i: (i, 0)),
            pl.BlockSpec((_TILE, 1), lambda i: (i, 0)),
            pl.BlockSpec((_TILE, _D), lambda i: (i, 0)),
            pl.BlockSpec((1, 1, _K), lambda i: (i, 0, 0)),
        ],
        out_shape=[
            jax.ShapeDtypeStruct((_N_TOKENS, _K), jnp.float32),
            jax.ShapeDtypeStruct((_N_TOKENS, 1), jnp.int32),
            jax.ShapeDtypeStruct((_N_TOKENS, _D), jnp.float32),
            jax.ShapeDtypeStruct((_GRID, 1, _K), jnp.float32),
        ],
        compiler_params=pltpu.CompilerParams(
            dimension_semantics=("arbitrary",),
        ),
    )(flat, W, b2, codebook, csq_x, a_x, jnp.ones((1, _TILE), jnp.float32))

    cb_pad = jnp.pad(codebook, ((0, 0), (0, _D_PAD - _D)))
    q = _make_gather()(cb_pad, idx.reshape(-1))[:, :_D]

    loss, perp = pl.pallas_call(
        _epilogue_body,
        out_shape=[
            jax.ShapeDtypeStruct((1, 1), jnp.float32),
            jax.ShapeDtypeStruct((1, 1), jnp.float32),
        ],
    )(q, z, cnt_part)

    return (loss[0, 0], q.reshape(B, S, _D), perp[0, 0], enc.reshape(B, S, _K))


# speedup vs baseline: 1.0359x; 1.0352x over previous
import functools

import jax
import jax.numpy as jnp
from jax import lax
from jax.experimental import pallas as pl
from jax.experimental.pallas import tpu as pltpu
from jax.experimental.pallas import tpu_sc as plsc

_INPUT_DIM = 96
_K = 8192
_D = 32
_BETA = 0.3
_N_TOKENS = 16 * 576
_TILE = 384
_GRID = _N_TOKENS // _TILE


def _main_body(x_ref, w_ref, b_ref, cb_ref, csq_ref, a_ref, ones_ref,
               enc_ref, idx_ref, z_ref, cnt_ref):
    x = x_ref[...]
    w = w_ref[...]
    b = b_ref[...]
    cb = cb_ref[...]
    csq = csq_ref[...]
    a = a_ref[...]
    z = jax.lax.dot_general(x, w, (((1,), (1,)), ((), ())),
                            preferred_element_type=jnp.float32) + b
    z2 = z + z
    m2 = jax.lax.dot_general(z2, cb, (((1,), (1,)), ((), ())),
                             preferred_element_type=jnp.float32)
    d = (a + csq) - m2

    n_chunks = 4
    cw = _K // n_chunks
    col = jax.lax.broadcasted_iota(jnp.int32, d.shape, 1)
    acc_v = None
    acc_i = None
    for c in range(n_chunks):
        dc = d[:, c * cw:(c + 1) * cw]
        vc = jnp.min(dc, axis=1, keepdims=True)
        wc = jnp.min(jnp.where(dc == vc, col[:, c * cw:(c + 1) * cw], _K),
                     axis=1, keepdims=True)
        vb = vc.astype(jnp.bfloat16).astype(jnp.float32)
        if acc_v is None:
            acc_v, acc_i = vb, wc
        else:
            take = vc < acc_v
            acc_i = jnp.where(take, wc, acc_i)
            acc_v = jnp.where(take, vb, acc_v)
    idx = acc_i

    onehot = (col == idx).astype(jnp.float32)
    enc_ref[...] = onehot
    idx_ref[...] = idx
    z_ref[...] = z
    cnt_ref[...] = jax.lax.dot_general(
        ones_ref[...], onehot, (((1,), (0,)), ((), ())),
        preferred_element_type=jnp.float32)[None]


def _epilogue_body(q_ref, z_ref, cnt_ref, loss_ref, perp_ref):
    q = q_ref[...]
    z = z_ref[...]
    diff = q - z
    mse = jnp.sum(diff * diff) / jnp.float32(_N_TOKENS * _D)
    loss_ref[...] = jnp.full((1, 1), mse + jnp.float32(_BETA) * mse,
                             dtype=jnp.float32)
    cnt = jnp.sum(cnt_ref[...][:, 0, :], axis=0, keepdims=True)
    p = cnt / jnp.float32(_N_TOKENS)
    perp = jnp.exp(-jnp.sum(p * jnp.log(p + 1e-10), keepdims=True))
    perp_ref[...] = perp.reshape(1, 1)


_D_PAD = 128


def _make_gather():
    info = plsc.get_sparse_core_info()
    num_cores = info.num_cores
    _NW = num_cores * info.num_subcores
    _B_PER_W = _N_TOKENS // _NW
    mesh = plsc.VectorSubcoreMesh(core_axis_name="c", subcore_axis_name="s")

    @functools.partial(
        pl.kernel, mesh=mesh,
        out_type=jax.ShapeDtypeStruct((_N_TOKENS, _D_PAD), jnp.float32),
        scratch_types=[
            pltpu.VMEM((_B_PER_W,), jnp.int32),
            pltpu.VMEM((_B_PER_W, _D_PAD), jnp.float32),
            pltpu.SemaphoreType.DMA,
        ],
    )
    def gather_k(table_hbm, idx_hbm, out_hbm, idx_v, rows_v, sem):
        wid = lax.axis_index("s") * num_cores + lax.axis_index("c")
        base = wid * _B_PER_W
        pltpu.sync_copy(idx_hbm.at[pl.ds(base, _B_PER_W)], idx_v)
        pltpu.async_copy(table_hbm.at[idx_v], rows_v, sem).wait()
        pltpu.sync_copy(rows_v, out_hbm.at[pl.ds(base, _B_PER_W)])

    return gather_k


def kernel(x, W, b, codebook):
    B, S, _ = x.shape
    flat = x.reshape(-1, _INPUT_DIM)
    b2 = b.reshape(1, _D)

    z_x = flat @ W.T + b
    a_x = jnp.sum(z_x ** 2, axis=1, keepdims=True)
    csq_x = jnp.sum(codebook ** 2, axis=1).reshape(1, _K)

    enc, idx, z, cnt_part = pl.pallas_call(
        _main_body,
        grid=(_GRID,),
        in_specs=[
            pl.BlockSpec((_TILE, _INPUT_DIM), lambda i: (i, 0)),
            pl.BlockSpec((_D, _INPUT_DIM), lambda i: (0, 0)),
            pl.BlockSpec((1, _D), lambda i: (0, 0)),
            pl.BlockSpec((_K, _D), lambda i: (0, 0)),
            pl.BlockSpec((1, _K), lambda i: (0, 0)),
            pl.BlockSpec((_TILE, 1), lambda i: (i, 0)),
            pl.BlockSpec((1, _TILE), lambda i: (0, 0)),
        ],
        out_specs=[
            pl.BlockSpec((_TILE, _K), lambda i: (i, 0)),
            pl.BlockSpec((_TILE, 1), lambda i: (i, 0)),
            pl.BlockSpec((_TILE, _D), lambda i: (i, 0)),
            pl.BlockSpec((1, 1, _K), lambda i: (i, 0, 0)),
        ],
        out_shape=[
            jax.ShapeDtypeStruct((_N_TOKENS, _K), jnp.float32),
            jax.ShapeDtypeStruct((_N_TOKENS, 1), jnp.int32),
            jax.ShapeDtypeStruct((_N_TOKENS, _D), jnp.float32),
            jax.ShapeDtypeStruct((_GRID, 1, _K), jnp.float32),
        ],
        compiler_params=pltpu.CompilerParams(
            dimension_semantics=("arbitrary",),
        ),
    )(flat, W, b2, codebook, csq_x, a_x, jnp.ones((1, _TILE), jnp.float32))

    cb_pad = jnp.pad(codebook, ((0, 0), (0, _D_PAD - _D)))
    q = _make_gather()(cb_pad, idx.reshape(-1))[:, :_D]

    loss, perp = pl.pallas_call(
        _epilogue_body,
        out_shape=[
            jax.ShapeDtypeStruct((1, 1), jnp.float32),
            jax.ShapeDtypeStruct((1, 1), jnp.float32),
        ],
    )(q, z, cnt_part)

    return (loss[0, 0], q.reshape(B, S, _D), perp[0, 0], enc.reshape(B, S, _K))
